# baseline (speedup 1.0000x reference)
import functools

import jax
import jax.numpy as jnp
from jax import lax
from jax.experimental import pallas as pl
from jax.experimental.pallas import tpu as pltpu
from jax.experimental.pallas import tpu_sc as plsc

_ORDER = 16
_L = 16
_NC = 2
_NS = 16
_NW = _NC * _NS
_ROWS = 200
_COLS = 16384
_CW = _COLS // _NW
_CR = 40
_NCHUNK = _ROWS // _CR
_NVEC = _CR * _CW // _L
_CVEC = _CW // _L


def _body(inp_hbm, table_hbm, out_hbm, table_v,
          in0, in1, out0, out1, si0, si1, so0, so1):
    wid = lax.axis_index("s") * _NC + lax.axis_index("c")
    col0 = wid * _CW
    pltpu.sync_copy(table_hbm, table_v)
    tbl = table_v[...]

    ins, outs = [in0, in1], [out0, out1]
    sin, sout = [si0, si1], [so0, so1]

    def start_in(ci):
        b = ci % 2
        return pltpu.async_copy(
            inp_hbm.at[pl.ds(ci * _CR, _CR), pl.ds(col0, _CW)], ins[b], sin[b])

    def start_out(ci):
        b = ci % 2
        return pltpu.async_copy(
            outs[b], out_hbm.at[pl.ds(ci * _CR, _CR), pl.ds(col0, _CW)],
            sout[b])

    in_copies = {0: start_in(0), 1: start_in(1)}
    out_copies = {}
    for ci in range(_NCHUNK):
        b = ci % 2
        in_copies[ci].wait()
        if ci >= 2:
            out_copies[ci - 2].wait()
        iv, ov = ins[b], outs[b]

        @plsc.parallel_loop(0, _CR, unroll=1)
        def _gather(r, iv=iv, ov=ov):
            for k in range(_CVEC):
                idx = iv[r, pl.ds(k * _L, _L)]
                ov[r, pl.ds(k * _L, _L)] = jnp.take_along_axis(
                    tbl, idx, axis=0, mode="promise_in_bounds")

        out_copies[ci] = start_out(ci)
        if ci + 2 < _NCHUNK:
            in_copies[ci + 2] = start_in(ci + 2)

    for ci in range(max(0, _NCHUNK - 2), _NCHUNK):
        out_copies[ci].wait()


def kernel(input, val_table):
    xt = input.T
    mesh = plsc.VectorSubcoreMesh(core_axis_name="c", subcore_axis_name="s")
    run = pl.kernel(
        _body,
        mesh=mesh,
        out_type=jax.ShapeDtypeStruct((_ROWS, _COLS), jnp.float32),
        scratch_types=[
            pltpu.VMEM((_ORDER,), jnp.float32),
            pltpu.VMEM((_CR, _CW), jnp.int32),
            pltpu.VMEM((_CR, _CW), jnp.int32),
            pltpu.VMEM((_CR, _CW), jnp.float32),
            pltpu.VMEM((_CR, _CW), jnp.float32),
            pltpu.SemaphoreType.DMA,
            pltpu.SemaphoreType.DMA,
            pltpu.SemaphoreType.DMA,
            pltpu.SemaphoreType.DMA,
        ],
        compiler_params=pltpu.CompilerParams(
            needs_layout_passes=False, use_tc_tiling_on_sc=True,
            skip_device_barrier=True),
    )
    return run(xt, val_table).T


# speedup vs baseline: 2.1259x; 2.1259x over previous
import jax
import jax.numpy as jnp
from jax.experimental import pallas as pl
from jax.experimental.pallas import tpu as pltpu

_ROWS = 200
_COLS = 16384
_BC = 2048


def _tc_body(tbl_ref, in_ref, out_ref):
    idx = in_ref[...]
    tblb = jnp.broadcast_to(tbl_ref[...][None, :], (_ROWS, 16))
    out_ref[...] = jnp.take_along_axis(
        tblb, idx, axis=1, mode="promise_in_bounds")


def tc_kernel(input, val_table):
    xt = input.T
    out = pl.pallas_call(
        _tc_body,
        grid=(_COLS // _BC,),
        in_specs=[
            pl.BlockSpec((16,), lambda i: (0,)),
            pl.BlockSpec((_ROWS, _BC), lambda i: (0, i)),
        ],
        out_specs=pl.BlockSpec((_ROWS, _BC), lambda i: (0, i)),
        out_shape=jax.ShapeDtypeStruct((_ROWS, _COLS), jnp.float32),
    )(val_table, xt)
    return out.T


def kernel(input, val_table):
    return tc_kernel(input, val_table)
